# baseline (speedup 1.0000x reference)
import jax
import jax.numpy as jnp
from jax import lax
from jax.experimental import pallas as pl
from jax.experimental.pallas import tpu as pltpu

NROWS = 32768
NCOLS = 1024
CH_R = 256
NCH = NROWS // CH_R
NBUF = 8
UNR = 8


def _tc_body(x_hbm, out_ref, *bufs_and_sems):
    bufs = bufs_and_sems[:NBUF]
    sems = bufs_and_sems[NBUF:]

    def copy(c, b):
        return pltpu.make_async_copy(
            x_hbm.at[pl.ds(c * CH_R, CH_R), :], bufs[b], sems[b])

    for b in range(NBUF):
        copy(b, b).start()

    def outer(c4, acc):
        base = c4 * NBUF
        for b in range(NBUF):
            c = base + b
            copy(c, b).wait()
            buf = bufs[b]

            def inner(r, acc, buf=buf):
                a = acc
                for u in range(UNR):
                    a = a + jnp.maximum(buf[pl.ds((r * UNR + u) * 8, 8), :], 0.0)
                return a

            acc = lax.fori_loop(0, CH_R // (8 * UNR), inner, acc)

            @pl.when(c4 < NCH // NBUF - 1)
            def _():
                copy(c + NBUF, b).start()

        return acc

    acc = lax.fori_loop(0, NCH // NBUF, outer,
                        jnp.zeros((8, NCOLS), jnp.float32))
    out_ref[0, 0] = jnp.sum(acc)


def kernel(x):
    tc_sum = pl.pallas_call(
        _tc_body,
        in_specs=[pl.BlockSpec(memory_space=pltpu.HBM)],
        out_specs=pl.BlockSpec(memory_space=pltpu.SMEM),
        out_shape=jax.ShapeDtypeStruct((1, 1), jnp.float32),
        scratch_shapes=(
            [pltpu.VMEM((CH_R, NCOLS), jnp.float32) for _ in range(NBUF)]
            + [pltpu.SemaphoreType.DMA for _ in range(NBUF)]
        ),
    )(x)
    return tc_sum[0, 0][None]


# speedup vs baseline: 1.0204x; 1.0204x over previous
import jax
import jax.numpy as jnp
from jax import lax
from jax.experimental import pallas as pl
from jax.experimental.pallas import tpu as pltpu

NROWS = 32768
NCOLS = 1024
CH_R = 256
NCH = NROWS // CH_R
NBUF = 8
UNR = 4


def _tc_body(x_hbm, out_ref, *bufs_and_sems):
    bufs = bufs_and_sems[:NBUF]
    sems = bufs_and_sems[NBUF:]

    def copy(c, b):
        return pltpu.make_async_copy(
            x_hbm.at[pl.ds(c * CH_R, CH_R), :], bufs[b], sems[b])

    for b in range(NBUF):
        copy(b, b).start()

    def outer(c4, acc):
        base = c4 * NBUF
        for b in range(NBUF):
            c = base + b
            copy(c, b).wait()
            buf = bufs[b]

            def inner(r, acc, buf=buf):
                a = acc
                for u in range(UNR):
                    a = a + jnp.maximum(buf[pl.ds((r * UNR + u) * 8, 8), :], 0.0)
                return a

            acc = lax.fori_loop(0, CH_R // (8 * UNR), inner, acc)

            @pl.when(c4 < NCH // NBUF - 1)
            def _():
                copy(c + NBUF, b).start()

        return acc

    acc = lax.fori_loop(0, NCH // NBUF, outer,
                        jnp.zeros((8, NCOLS), jnp.float32))
    out_ref[0, 0] = jnp.sum(acc)


def kernel(x):
    tc_sum = pl.pallas_call(
        _tc_body,
        in_specs=[pl.BlockSpec(memory_space=pltpu.HBM)],
        out_specs=pl.BlockSpec(memory_space=pltpu.SMEM),
        out_shape=jax.ShapeDtypeStruct((1, 1), jnp.float32),
        scratch_shapes=(
            [pltpu.VMEM((CH_R, NCOLS), jnp.float32) for _ in range(NBUF)]
            + [pltpu.SemaphoreType.DMA for _ in range(NBUF)]
        ),
    )(x)
    return tc_sum[0, 0][None]
